# baseline (speedup 1.0000x reference)
import functools

import jax
import jax.numpy as jnp
from jax import lax
from jax.experimental import pallas as pl
from jax.experimental.pallas import tpu as pltpu
from jax.experimental.pallas import tpu_sc as plsc

N = 10000
H = 128
G = 64

NC = 2
NS = 16
NW = NC * NS

J = 4
NH = N // NC
RPA = (NH // NS) // 8 * 8
TAILA = NH - NS * RPA
NP = N // 2
RPU = (NP // NS) // 8 * 8
TAILU = NP - NS * RPU


def _sc_deg_body(ew_hbm, dst_hbm, out_hbm, ewb, dstb, degloc):
    rows_per_w = ew_hbm.shape[0] // NW
    c = lax.axis_index("c")
    s = lax.axis_index("s")
    wid = s * NC + c

    def zb(i, carry):
        degloc[0, pl.ds(i * 16, 16)] = jnp.zeros((16,), jnp.float32)
        return carry
    lax.fori_loop(0, N // 16, zb, 0)

    r0 = wid * rows_per_w
    pltpu.sync_copy(ew_hbm.at[pl.ds(r0, rows_per_w)], ewb)
    pltpu.sync_copy(dst_hbm.at[pl.ds(r0, rows_per_w)], dstb)

    zero16 = jnp.zeros((16,), jnp.int32)

    def eb(g, carry):
        r = g // 8
        q = (g % 8) * 16
        idx = dstb[r, pl.ds(q, 16)]
        vals = ewb[r, pl.ds(q, 16)]
        plsc.addupdate_scatter(degloc, [zero16, idx], vals)
        return carry
    lax.fori_loop(0, rows_per_w * 8, eb, 0)

    pltpu.sync_copy(degloc, out_hbm.at[wid])


def _sc_edges_body(ut_hbm, src_hbm, dst_hbm, ew_hbm, out_hbm,
                   isrc, idst, ewb, gb0, gb1, fb, usp, accum, gsem, ssem):
    nchunk = src_hbm.shape[0] // (NS * J)
    c = lax.axis_index("c")
    s = lax.axis_index("s")
    lo = c * NH
    gbufs = (gb0, gb1)

    def zrow(rw, carry):
        for q in range(8):
            fb[rw, pl.ds(q * 16, 16)] = jnp.zeros((16,), jnp.float32)
        return carry
    lax.fori_loop(0, 128, zrow, 0)
    oa = s * RPA
    off = 0
    while off < RPA:
        sz = min(128, RPA - off)
        pltpu.sync_copy(fb.at[pl.ds(0, sz)], accum.at[pl.ds(oa + off, sz)])
        off += sz

    ou = s * RPU
    off = 0
    while off < RPU:
        sz = min(128, RPU - off)
        pltpu.sync_copy(ut_hbm.at[pl.ds(ou + off, sz)], gb0.at[pl.ds(0, sz)])
        pltpu.sync_copy(gb0.at[pl.ds(0, sz)], usp.at[pl.ds(ou + off, sz)])
        off += sz

    @pl.when(s == NS - 1)
    def _():
        pltpu.sync_copy(fb.at[pl.ds(0, TAILA)],
                        accum.at[pl.ds(NS * RPA, TAILA)])
        pltpu.sync_copy(ut_hbm.at[pl.ds(NS * RPU, TAILU)],
                        gb0.at[pl.ds(0, TAILU)])
        pltpu.sync_copy(gb0.at[pl.ds(0, TAILU)], usp.at[pl.ds(NS * RPU, TAILU)])
    plsc.subcore_barrier()

    base_row = s * (nchunk * J)

    def chunk_body(g, carry):
        r0 = base_row + g * J
        pltpu.sync_copy(src_hbm.at[pl.ds(r0, J)], isrc)
        pltpu.sync_copy(dst_hbm.at[pl.ds(r0, J)], idst)
        pltpu.sync_copy(ew_hbm.at[pl.ds(r0, J)], ewb)

        def prep(i, carry2):
            j = i // 8
            q = (i % 8) * 16
            sv16 = isrc[j, pl.ds(q, 16)]
            dv = idst[j, pl.ds(q, 16)]
            ev = ewb[j, pl.ds(q, 16)]
            m = (dv >= lo) & (dv < lo + NH)
            sgn = 1.0 - 2.0 * (sv16 & 1).astype(jnp.float32)
            ewb[j, pl.ds(q, 16)] = jnp.where(m, ev, 0.0) * sgn
            idst[j, pl.ds(q, 16)] = jnp.clip(dv - lo, 0, NH - 1)
            isrc[j, pl.ds(q, 16)] = sv16 >> 1
            return carry2
        lax.fori_loop(0, J * 8, prep, 0)

        pltpu.async_copy(usp.at[isrc.at[0]], gbufs[0], gsem)
        for j in range(J):
            gb = gbufs[j % 2]
            pltpu.make_async_copy(usp.at[isrc.at[j]], gb, gsem).wait()
            if j + 1 < J:
                pltpu.async_copy(usp.at[isrc.at[j + 1]], gbufs[1 - j % 2], gsem)
            if j >= 1:
                pltpu.make_async_copy(fb, accum.at[idst.at[j - 1]], ssem).wait()

            @plsc.parallel_loop(0, 128, step=1, unroll=4)
            def _(t):
                idx_j = jnp.full((16,), j, jnp.int32)
                idx_t = jnp.full((16,), t, jnp.int32)
                ev = plsc.load_gather(ewb, [idx_j, idx_t])
                sv = jnp.abs(ev)
                coladd = jnp.where(ev < 0.0, 64, 0).astype(jnp.int32)
                it16 = lax.iota(jnp.int32, 16)
                for q in range(4):
                    cols = coladd + (16 * q) + it16
                    w = plsc.load_gather(gb, [idx_t, cols])
                    v = plsc.bitcast(w, jnp.bfloat16)
                    a, b = plsc.unpack(v, format=plsc.PackFormat.INTERLEAVED)
                    fb[t, pl.ds(32 * q, 16)] = a * sv
                    fb[t, pl.ds(32 * q + 16, 16)] = b * sv

            pltpu.async_copy(fb, accum.at[idst.at[j]], ssem, add=True)
        pltpu.make_async_copy(fb, accum.at[idst.at[J - 1]], ssem).wait()
        return carry
    lax.fori_loop(0, nchunk, chunk_body, 0)

    plsc.subcore_barrier()
    off = 0
    while off < RPA:
        sz = min(128, RPA - off)
        pltpu.sync_copy(accum.at[pl.ds(oa + off, sz)],
                        out_hbm.at[pl.ds(lo + oa + off, sz)])
        off += sz

    @pl.when(s == NS - 1)
    def _():
        pltpu.sync_copy(accum.at[pl.ds(NS * RPA, TAILA)],
                        out_hbm.at[pl.ds(lo + NS * RPA, TAILA)])


def _tc_dinv_body(dp_ref, dinv_ref):
    deg = jnp.sum(dp_ref[...], axis=0) + 1.0
    dinv_ref[...] = jnp.where(deg > 0.0, lax.rsqrt(deg), 0.0)


def _tc_pre_body(dinv_ref, x_ref, w_ref, wt_ref, xw_ref, ut_ref):
    dinv = dinv_ref[0, 0, :]
    xw = jnp.dot(x_ref[...], w_ref[...], preferred_element_type=jnp.float32)
    xw_ref[...] = xw
    xwt = jnp.dot(x_ref[...], wt_ref[...], preferred_element_type=jnp.float32)
    ut_ref[...] = (xwt * dinv[:, None]).astype(jnp.bfloat16)


def _tc_mid_body(p_ref, xw_ref, dinv_ref, b_ref, w_ref, wt_ref,
                 xw_out_ref, ut_out_ref):
    dinv = dinv_ref[0, 0, :]
    h = jnp.maximum(
        p_ref[...] * dinv[:, None] + xw_ref[...] * (dinv * dinv)[:, None]
        + b_ref[0][None, :], 0.0)
    xw2 = jnp.dot(h, w_ref[...], preferred_element_type=jnp.float32)
    xw_out_ref[...] = xw2
    xwt = jnp.dot(h, wt_ref[...], preferred_element_type=jnp.float32)
    ut_out_ref[...] = (xwt * dinv[:, None]).astype(jnp.bfloat16)


def _tc_pool_body(p_ref, xw_ref, dinv_ref, b_ref, batch_ref, fcw_ref, fcb_ref,
                  out_ref, sums_ref, cnts_ref):
    i = pl.program_id(0)

    @pl.when(i == 0)
    def _():
        sums_ref[...] = jnp.zeros_like(sums_ref)
        cnts_ref[...] = jnp.zeros_like(cnts_ref)

    dinv = dinv_ref[0, 0, :]
    h = jnp.maximum(
        p_ref[...] * dinv[:, None] + xw_ref[...] * (dinv * dinv)[:, None]
        + b_ref[0][None, :], 0.0)
    seg = batch_ref[0, 0, :]
    bn = seg.shape[0]
    onehot = (lax.broadcasted_iota(jnp.int32, (G, bn), 0)
              == seg[None, :]).astype(jnp.float32)
    sums_ref[...] += jnp.dot(onehot, h, preferred_element_type=jnp.float32)
    cnts_ref[...] += jnp.sum(onehot, axis=1, keepdims=True)

    @pl.when(i == pl.num_programs(0) - 1)
    def _():
        pooled = sums_ref[...] / jnp.maximum(cnts_ref[...], 1.0)
        res = jnp.dot(pooled, fcw_ref[...], preferred_element_type=jnp.float32)
        out_ref[...] = res[:, 0] + fcb_ref[0, 0]


def _mesh():
    return plsc.VectorSubcoreMesh(core_axis_name="c", subcore_axis_name="s",
                                  num_cores=NC, num_subcores=NS)


_SC_PARAMS = pltpu.CompilerParams(needs_layout_passes=False)


@jax.jit
def kernel(x, edge_index, edge_attr, batch, W1, b1, W2, b2, W3, b3, fcW, fcb):
    E = edge_attr.shape[0]
    unit = NS * J * 128
    epad = ((E + unit - 1) // unit) * unit
    pad = epad - E
    erows = epad // 128

    src = edge_index[0].astype(jnp.int32)
    dst = edge_index[1].astype(jnp.int32)
    ew = edge_attr.astype(jnp.float32)
    if pad:
        zi = jnp.zeros((pad,), jnp.int32)
        src = jnp.concatenate([src, zi])
        dst = jnp.concatenate([dst, zi])
        ew = jnp.concatenate([ew, jnp.zeros((pad,), jnp.float32)])
    src2 = src.reshape(erows, 128)
    dst2 = dst.reshape(erows, 128)
    ew2 = ew.reshape(erows, 128)

    k = jnp.arange(H)
    grp, r = k // 32, k % 32
    tau = 32 * grp + jnp.where(r % 2 == 0, r // 2, 16 + r // 2)
    W1t = W1[:, tau]
    W2t = W2[:, tau]
    W3t = W3[:, tau]

    sc_deg = pl.kernel(
        _sc_deg_body,
        out_type=jax.ShapeDtypeStruct((NW, 1, N), jnp.float32),
        mesh=_mesh(),
        scratch_types=[
            pltpu.VMEM((erows // NW, 128), jnp.float32),
            pltpu.VMEM((erows // NW, 128), jnp.int32),
            pltpu.VMEM((1, N), jnp.float32),
        ],
        compiler_params=_SC_PARAMS,
    )
    deg_partials = sc_deg(ew2, dst2)

    dinv2d = pl.pallas_call(
        _tc_dinv_body,
        grid=(1,),
        in_specs=[pl.BlockSpec((NW, 1, N), lambda i: (0, 0, 0))],
        out_specs=pl.BlockSpec((1, N), lambda i: (0, 0)),
        out_shape=jax.ShapeDtypeStruct((1, N), jnp.float32),
    )(deg_partials)

    nb = 5
    bn = N // nb
    dinv3 = dinv2d.reshape(nb, 1, bn)
    batch3 = batch.astype(jnp.int32).reshape(nb, 1, bn)

    row_spec = pl.BlockSpec((bn, H), lambda i: (i, 0))
    dinv_spec = pl.BlockSpec((1, 1, bn), lambda i: (i, 0, 0))
    w_spec = pl.BlockSpec((H, H), lambda i: (0, 0))
    b_spec = pl.BlockSpec((1, H), lambda i: (0, 0))
    mat_shape = jax.ShapeDtypeStruct((N, H), jnp.float32)
    ut_shape = jax.ShapeDtypeStruct((N, H), jnp.bfloat16)

    tc_pre = pl.pallas_call(
        _tc_pre_body,
        grid=(nb,),
        in_specs=[dinv_spec, row_spec, w_spec, w_spec],
        out_specs=[row_spec, row_spec],
        out_shape=[mat_shape, ut_shape],
    )

    tc_mid = pl.pallas_call(
        _tc_mid_body,
        grid=(nb,),
        in_specs=[row_spec, row_spec, dinv_spec, b_spec, w_spec, w_spec],
        out_specs=[row_spec, row_spec],
        out_shape=[mat_shape, ut_shape],
    )

    sc_edges = pl.kernel(
        _sc_edges_body,
        out_type=jax.ShapeDtypeStruct((N, H), jnp.float32),
        mesh=_mesh(),
        scratch_types=[
            pltpu.VMEM((J, 128), jnp.int32),
            pltpu.VMEM((J, 128), jnp.int32),
            pltpu.VMEM((J, 128), jnp.float32),
            pltpu.VMEM((128, 128), jnp.int32),
            pltpu.VMEM((128, 128), jnp.int32),
            pltpu.VMEM((128, H), jnp.float32),
            pltpu.VMEM_SHARED((NP, 128), jnp.int32),
            pltpu.VMEM_SHARED((NH, H), jnp.float32),
            pltpu.SemaphoreType.DMA,
            pltpu.SemaphoreType.DMA,
        ],
        compiler_params=_SC_PARAMS,
    )

    def pack_i32(ut):
        return lax.bitcast_convert_type(
            ut.reshape(NP, 128, 2), jnp.int32)

    xw1, ut1 = tc_pre(dinv3, x, W1, W1t)
    p1 = sc_edges(pack_i32(ut1), src2, dst2, ew2)
    xw2, ut2 = tc_mid(p1, xw1, dinv3, b1.reshape(1, H), W2, W2t)
    p2 = sc_edges(pack_i32(ut2), src2, dst2, ew2)
    xw3, ut3 = tc_mid(p2, xw2, dinv3, b2.reshape(1, H), W3, W3t)
    p3 = sc_edges(pack_i32(ut3), src2, dst2, ew2)

    out, _sums, _cnts = pl.pallas_call(
        _tc_pool_body,
        grid=(nb,),
        in_specs=[row_spec, row_spec, dinv_spec, b_spec,
                  pl.BlockSpec((1, 1, bn), lambda i: (i, 0, 0)),
                  pl.BlockSpec((H, 1), lambda i: (0, 0)),
                  pl.BlockSpec((1, 1), lambda i: (0, 0))],
        out_specs=[pl.BlockSpec((G,), lambda i: (0,)),
                   pl.BlockSpec((G, H), lambda i: (0, 0)),
                   pl.BlockSpec((G, 1), lambda i: (0, 0))],
        out_shape=[jax.ShapeDtypeStruct((G,), jnp.float32),
                   jax.ShapeDtypeStruct((G, H), jnp.float32),
                   jax.ShapeDtypeStruct((G, 1), jnp.float32)],
    )(p3, xw3, dinv3, b3.reshape(1, H), batch3, fcW, fcb.reshape(1, 1))

    return out


# speedup vs baseline: 1.0033x; 1.0033x over previous
import functools

import jax
import jax.numpy as jnp
from jax import lax
from jax.experimental import pallas as pl
from jax.experimental.pallas import tpu as pltpu
from jax.experimental.pallas import tpu_sc as plsc

N = 10000
H = 128
G = 64

NC = 2
NS = 16
NW = NC * NS

J = 4
NH = N // NC
RPA = (NH // NS) // 8 * 8
TAILA = NH - NS * RPA
NP = N // 2
RPU = (NP // NS) // 8 * 8
TAILU = NP - NS * RPU


def _sc_deg_body(ew_hbm, dst_hbm, out_hbm, ewb, dstb, degloc):
    rows_per_w = ew_hbm.shape[0] // NW
    c = lax.axis_index("c")
    s = lax.axis_index("s")
    wid = s * NC + c

    def zb(i, carry):
        degloc[0, pl.ds(i * 16, 16)] = jnp.zeros((16,), jnp.float32)
        return carry
    lax.fori_loop(0, N // 16, zb, 0)

    r0 = wid * rows_per_w
    pltpu.sync_copy(ew_hbm.at[pl.ds(r0, rows_per_w)], ewb)
    pltpu.sync_copy(dst_hbm.at[pl.ds(r0, rows_per_w)], dstb)

    zero16 = jnp.zeros((16,), jnp.int32)

    def eb(g, carry):
        r = g // 8
        q = (g % 8) * 16
        idx = dstb[r, pl.ds(q, 16)]
        vals = ewb[r, pl.ds(q, 16)]
        plsc.addupdate_scatter(degloc, [zero16, idx], vals)
        return carry
    lax.fori_loop(0, rows_per_w * 8, eb, 0)

    pltpu.sync_copy(degloc, out_hbm.at[wid])


def _sc_edges_body(ut_hbm, src_hbm, dst_hbm, ew_hbm, out_hbm,
                   isrc, idst, ewb, gb0, gb1, fb, usp, accum, gsem, ssem):
    nchunk = src_hbm.shape[0] // (NS * J)
    c = lax.axis_index("c")
    s = lax.axis_index("s")
    lo = c * NH
    gbufs = (gb0, gb1)

    def zrow(rw, carry):
        for q in range(8):
            fb[rw, pl.ds(q * 16, 16)] = jnp.zeros((16,), jnp.float32)
        return carry
    lax.fori_loop(0, 128, zrow, 0)
    oa = s * RPA
    off = 0
    while off < RPA:
        sz = min(128, RPA - off)
        pltpu.sync_copy(fb.at[pl.ds(0, sz)], accum.at[pl.ds(oa + off, sz)])
        off += sz

    ou = s * RPU
    off = 0
    while off < RPU:
        sz = min(128, RPU - off)
        pltpu.sync_copy(ut_hbm.at[pl.ds(ou + off, sz)], gb0.at[pl.ds(0, sz)])
        pltpu.sync_copy(gb0.at[pl.ds(0, sz)], usp.at[pl.ds(ou + off, sz)])
        off += sz

    @pl.when(s == NS - 1)
    def _():
        pltpu.sync_copy(fb.at[pl.ds(0, TAILA)],
                        accum.at[pl.ds(NS * RPA, TAILA)])
        pltpu.sync_copy(ut_hbm.at[pl.ds(NS * RPU, TAILU)],
                        gb0.at[pl.ds(0, TAILU)])
        pltpu.sync_copy(gb0.at[pl.ds(0, TAILU)], usp.at[pl.ds(NS * RPU, TAILU)])
    plsc.subcore_barrier()

    base_row = s * (nchunk * J)

    def chunk_body(g, carry):
        r0 = base_row + g * J
        pltpu.sync_copy(src_hbm.at[pl.ds(r0, J)], isrc)
        pltpu.sync_copy(dst_hbm.at[pl.ds(r0, J)], idst)
        pltpu.sync_copy(ew_hbm.at[pl.ds(r0, J)], ewb)

        def prep(i, carry2):
            j = i // 8
            q = (i % 8) * 16
            sv16 = isrc[j, pl.ds(q, 16)]
            dv = idst[j, pl.ds(q, 16)]
            ev = ewb[j, pl.ds(q, 16)]
            m = (dv >= lo) & (dv < lo + NH)
            sgn = 1.0 - 2.0 * (sv16 & 1).astype(jnp.float32)
            ewb[j, pl.ds(q, 16)] = jnp.where(m, ev, 0.0) * sgn
            idst[j, pl.ds(q, 16)] = jnp.clip(dv - lo, 0, NH - 1)
            isrc[j, pl.ds(q, 16)] = sv16 >> 1
            return carry2
        lax.fori_loop(0, J * 8, prep, 0)

        pltpu.async_copy(usp.at[isrc.at[0]], gbufs[0], gsem)
        for j in range(J):
            gb = gbufs[j % 2]
            pltpu.make_async_copy(usp.at[isrc.at[j]], gb, gsem).wait()
            if j + 1 < J:
                pltpu.async_copy(usp.at[isrc.at[j + 1]], gbufs[1 - j % 2], gsem)
            if j >= 1:
                pltpu.make_async_copy(fb, accum.at[idst.at[j - 1]], ssem).wait()

            @plsc.parallel_loop(0, 128, step=1, unroll=4)
            def _(t):
                idx_j = jnp.full((16,), j, jnp.int32)
                idx_t = jnp.full((16,), t, jnp.int32)
                ev = plsc.load_gather(ewb, [idx_j, idx_t])
                sv = jnp.abs(ev)
                coladd = jnp.where(ev < 0.0, 64, 0).astype(jnp.int32)
                it16 = lax.iota(jnp.int32, 16)
                for q in range(4):
                    cols = coladd + (16 * q) + it16
                    w = plsc.load_gather(gb, [idx_t, cols])
                    a = plsc.bitcast(w << 16, jnp.float32)
                    b = plsc.bitcast(
                        w & jnp.int32(-65536), jnp.float32)
                    fb[t, pl.ds(32 * q, 16)] = a * sv
                    fb[t, pl.ds(32 * q + 16, 16)] = b * sv

            pltpu.async_copy(fb, accum.at[idst.at[j]], ssem, add=True)
        pltpu.make_async_copy(fb, accum.at[idst.at[J - 1]], ssem).wait()
        return carry
    lax.fori_loop(0, nchunk, chunk_body, 0)

    plsc.subcore_barrier()
    off = 0
    while off < RPA:
        sz = min(128, RPA - off)
        pltpu.sync_copy(accum.at[pl.ds(oa + off, sz)],
                        out_hbm.at[pl.ds(lo + oa + off, sz)])
        off += sz

    @pl.when(s == NS - 1)
    def _():
        pltpu.sync_copy(accum.at[pl.ds(NS * RPA, TAILA)],
                        out_hbm.at[pl.ds(lo + NS * RPA, TAILA)])


def _tc_dinv_body(dp_ref, dinv_ref):
    deg = jnp.sum(dp_ref[...], axis=0) + 1.0
    dinv_ref[...] = jnp.where(deg > 0.0, lax.rsqrt(deg), 0.0)


def _tc_pre_body(dinv_ref, x_ref, w_ref, wt_ref, xw_ref, ut_ref):
    dinv = dinv_ref[0, 0, :]
    xw = jnp.dot(x_ref[...], w_ref[...], preferred_element_type=jnp.float32)
    xw_ref[...] = xw
    xwt = jnp.dot(x_ref[...], wt_ref[...], preferred_element_type=jnp.float32)
    ut_ref[...] = (xwt * dinv[:, None]).astype(jnp.bfloat16)


def _tc_mid_body(p_ref, xw_ref, dinv_ref, b_ref, w_ref, wt_ref,
                 xw_out_ref, ut_out_ref):
    dinv = dinv_ref[0, 0, :]
    h = jnp.maximum(
        p_ref[...] * dinv[:, None] + xw_ref[...] * (dinv * dinv)[:, None]
        + b_ref[0][None, :], 0.0)
    xw2 = jnp.dot(h, w_ref[...], preferred_element_type=jnp.float32)
    xw_out_ref[...] = xw2
    xwt = jnp.dot(h, wt_ref[...], preferred_element_type=jnp.float32)
    ut_out_ref[...] = (xwt * dinv[:, None]).astype(jnp.bfloat16)


def _tc_pool_body(p_ref, xw_ref, dinv_ref, b_ref, batch_ref, fcw_ref, fcb_ref,
                  out_ref, sums_ref, cnts_ref):
    i = pl.program_id(0)

    @pl.when(i == 0)
    def _():
        sums_ref[...] = jnp.zeros_like(sums_ref)
        cnts_ref[...] = jnp.zeros_like(cnts_ref)

    dinv = dinv_ref[0, 0, :]
    h = jnp.maximum(
        p_ref[...] * dinv[:, None] + xw_ref[...] * (dinv * dinv)[:, None]
        + b_ref[0][None, :], 0.0)
    seg = batch_ref[0, 0, :]
    bn = seg.shape[0]
    onehot = (lax.broadcasted_iota(jnp.int32, (G, bn), 0)
              == seg[None, :]).astype(jnp.float32)
    sums_ref[...] += jnp.dot(onehot, h, preferred_element_type=jnp.float32)
    cnts_ref[...] += jnp.sum(onehot, axis=1, keepdims=True)

    @pl.when(i == pl.num_programs(0) - 1)
    def _():
        pooled = sums_ref[...] / jnp.maximum(cnts_ref[...], 1.0)
        res = jnp.dot(pooled, fcw_ref[...], preferred_element_type=jnp.float32)
        out_ref[...] = res[:, 0] + fcb_ref[0, 0]


def _mesh():
    return plsc.VectorSubcoreMesh(core_axis_name="c", subcore_axis_name="s",
                                  num_cores=NC, num_subcores=NS)


_SC_PARAMS = pltpu.CompilerParams(needs_layout_passes=False)


@jax.jit
def kernel(x, edge_index, edge_attr, batch, W1, b1, W2, b2, W3, b3, fcW, fcb):
    E = edge_attr.shape[0]
    unit = NS * J * 128
    epad = ((E + unit - 1) // unit) * unit
    pad = epad - E
    erows = epad // 128

    src = edge_index[0].astype(jnp.int32)
    dst = edge_index[1].astype(jnp.int32)
    ew = edge_attr.astype(jnp.float32)
    if pad:
        zi = jnp.zeros((pad,), jnp.int32)
        src = jnp.concatenate([src, zi])
        dst = jnp.concatenate([dst, zi])
        ew = jnp.concatenate([ew, jnp.zeros((pad,), jnp.float32)])
    src2 = src.reshape(erows, 128)
    dst2 = dst.reshape(erows, 128)
    ew2 = ew.reshape(erows, 128)

    k = jnp.arange(H)
    grp, r = k // 32, k % 32
    tau = 32 * grp + jnp.where(r % 2 == 0, r // 2, 16 + r // 2)
    W1t = W1[:, tau]
    W2t = W2[:, tau]
    W3t = W3[:, tau]

    sc_deg = pl.kernel(
        _sc_deg_body,
        out_type=jax.ShapeDtypeStruct((NW, 1, N), jnp.float32),
        mesh=_mesh(),
        scratch_types=[
            pltpu.VMEM((erows // NW, 128), jnp.float32),
            pltpu.VMEM((erows // NW, 128), jnp.int32),
            pltpu.VMEM((1, N), jnp.float32),
        ],
        compiler_params=_SC_PARAMS,
    )
    deg_partials = sc_deg(ew2, dst2)

    dinv2d = pl.pallas_call(
        _tc_dinv_body,
        grid=(1,),
        in_specs=[pl.BlockSpec((NW, 1, N), lambda i: (0, 0, 0))],
        out_specs=pl.BlockSpec((1, N), lambda i: (0, 0)),
        out_shape=jax.ShapeDtypeStruct((1, N), jnp.float32),
    )(deg_partials)

    nb = 5
    bn = N // nb
    dinv3 = dinv2d.reshape(nb, 1, bn)
    batch3 = batch.astype(jnp.int32).reshape(nb, 1, bn)

    row_spec = pl.BlockSpec((bn, H), lambda i: (i, 0))
    dinv_spec = pl.BlockSpec((1, 1, bn), lambda i: (i, 0, 0))
    w_spec = pl.BlockSpec((H, H), lambda i: (0, 0))
    b_spec = pl.BlockSpec((1, H), lambda i: (0, 0))
    mat_shape = jax.ShapeDtypeStruct((N, H), jnp.float32)
    ut_shape = jax.ShapeDtypeStruct((N, H), jnp.bfloat16)

    tc_pre = pl.pallas_call(
        _tc_pre_body,
        grid=(nb,),
        in_specs=[dinv_spec, row_spec, w_spec, w_spec],
        out_specs=[row_spec, row_spec],
        out_shape=[mat_shape, ut_shape],
    )

    tc_mid = pl.pallas_call(
        _tc_mid_body,
        grid=(nb,),
        in_specs=[row_spec, row_spec, dinv_spec, b_spec, w_spec, w_spec],
        out_specs=[row_spec, row_spec],
        out_shape=[mat_shape, ut_shape],
    )

    sc_edges = pl.kernel(
        _sc_edges_body,
        out_type=jax.ShapeDtypeStruct((N, H), jnp.float32),
        mesh=_mesh(),
        scratch_types=[
            pltpu.VMEM((J, 128), jnp.int32),
            pltpu.VMEM((J, 128), jnp.int32),
            pltpu.VMEM((J, 128), jnp.float32),
            pltpu.VMEM((128, 128), jnp.int32),
            pltpu.VMEM((128, 128), jnp.int32),
            pltpu.VMEM((128, H), jnp.float32),
            pltpu.VMEM_SHARED((NP, 128), jnp.int32),
            pltpu.VMEM_SHARED((NH, H), jnp.float32),
            pltpu.SemaphoreType.DMA,
            pltpu.SemaphoreType.DMA,
        ],
        compiler_params=_SC_PARAMS,
    )

    def pack_i32(ut):
        return lax.bitcast_convert_type(
            ut.reshape(NP, 128, 2), jnp.int32)

    xw1, ut1 = tc_pre(dinv3, x, W1, W1t)
    p1 = sc_edges(pack_i32(ut1), src2, dst2, ew2)
    xw2, ut2 = tc_mid(p1, xw1, dinv3, b1.reshape(1, H), W2, W2t)
    p2 = sc_edges(pack_i32(ut2), src2, dst2, ew2)
    xw3, ut3 = tc_mid(p2, xw2, dinv3, b2.reshape(1, H), W3, W3t)
    p3 = sc_edges(pack_i32(ut3), src2, dst2, ew2)

    out, _sums, _cnts = pl.pallas_call(
        _tc_pool_body,
        grid=(nb,),
        in_specs=[row_spec, row_spec, dinv_spec, b_spec,
                  pl.BlockSpec((1, 1, bn), lambda i: (i, 0, 0)),
                  pl.BlockSpec((H, 1), lambda i: (0, 0)),
                  pl.BlockSpec((1, 1), lambda i: (0, 0))],
        out_specs=[pl.BlockSpec((G,), lambda i: (0,)),
                   pl.BlockSpec((G, H), lambda i: (0, 0)),
                   pl.BlockSpec((G, 1), lambda i: (0, 0))],
        out_shape=[jax.ShapeDtypeStruct((G,), jnp.float32),
                   jax.ShapeDtypeStruct((G, H), jnp.float32),
                   jax.ShapeDtypeStruct((G, 1), jnp.float32)],
    )(p3, xw3, dinv3, b3.reshape(1, H), batch3, fcW, fcb.reshape(1, 1))

    return out


# speedup vs baseline: 1.7864x; 1.7806x over previous
import functools

import jax
import jax.numpy as jnp
from jax import lax
from jax.experimental import pallas as pl
from jax.experimental.pallas import tpu as pltpu
from jax.experimental.pallas import tpu_sc as plsc

N = 10000
H = 128
G = 64

NC = 2
NS = 16
NW = NC * NS

J = 8
JH = 2
K = JH * 128
RPT = (N // NS) // 8 * 8
TAIL = N - NS * RPT


def _sc_deg_body(ew_hbm, dst_hbm, out_hbm, ewb, dstb, degloc):
    rows_per_w = ew_hbm.shape[0] // NW
    c = lax.axis_index("c")
    s = lax.axis_index("s")
    wid = s * NC + c

    def zb(i, carry):
        degloc[0, pl.ds(i * 16, 16)] = jnp.zeros((16,), jnp.float32)
        return carry
    lax.fori_loop(0, N // 16, zb, 0)

    r0 = wid * rows_per_w
    pltpu.sync_copy(ew_hbm.at[pl.ds(r0, rows_per_w)], ewb)
    pltpu.sync_copy(dst_hbm.at[pl.ds(r0, rows_per_w)], dstb)

    zero16 = jnp.zeros((16,), jnp.int32)

    def eb(g, carry):
        r = g // 8
        q = (g % 8) * 16
        idx = dstb[r, pl.ds(q, 16)]
        vals = ewb[r, pl.ds(q, 16)]
        plsc.addupdate_scatter(degloc, [zero16, idx], vals)
        return carry
    lax.fori_loop(0, rows_per_w * 8, eb, 0)

    pltpu.sync_copy(degloc, out_hbm.at[wid])


def _sc_edges_body(u_hbm, src_hbm, dst_hbm, ew_hbm, out_hbm,
                   isrc, idst, ewb, rows0, rows1, accum, gsem, ssem):
    nchunk = src_hbm.shape[0] // (NW * J)
    c = lax.axis_index("c")
    s = lax.axis_index("s")
    wid = s * NC + c
    rows = (rows0, rows1)

    def zrow(rw, carry):
        for q in range(8):
            rows0[rw, pl.ds(q * 16, 16)] = jnp.zeros((16,), jnp.float32)
        return carry
    lax.fori_loop(0, 128, zrow, 0)
    o0 = s * RPT
    off = 0
    while off < RPT:
        sz = min(128, RPT - off)
        pltpu.sync_copy(rows0.at[pl.ds(0, sz)], accum.at[pl.ds(o0 + off, sz)])
        off += sz

    @pl.when(s == NS - 1)
    def _():
        pltpu.sync_copy(rows0.at[pl.ds(0, TAIL)],
                        accum.at[pl.ds(NS * RPT, TAIL)])
    plsc.subcore_barrier()

    base_row = wid * (nchunk * J)

    def chunk_body(g, carry):
        r0 = base_row + g * J
        pltpu.sync_copy(src_hbm.at[pl.ds(r0, J)], isrc)
        pltpu.sync_copy(dst_hbm.at[pl.ds(r0, J)], idst)
        pltpu.sync_copy(ew_hbm.at[pl.ds(r0, J)], ewb)
        pltpu.async_copy(u_hbm.at[isrc.at[0]], rows[0], gsem)
        for j in range(J):
            b = j % 2
            rb = rows[b]
            pltpu.make_async_copy(u_hbm.at[isrc.at[j]], rb, gsem).wait()
            if j + 1 < J:
                if j >= 1:
                    pltpu.make_async_copy(rows[1 - b],
                                          accum.at[idst.at[j - 1]],
                                          ssem).wait()
                pltpu.async_copy(u_hbm.at[isrc.at[j + 1]], rows[1 - b], gsem)

            @plsc.parallel_loop(0, 128, step=1, unroll=4)
            def _(t):
                idx_j = jnp.full((16,), j, jnp.int32)
                idx_t = jnp.full((16,), t, jnp.int32)
                sv = plsc.load_gather(ewb, [idx_j, idx_t])
                for q in range(8):
                    rb[t, pl.ds(q * 16, 16)] = rb[t, pl.ds(q * 16, 16)] * sv

            pltpu.async_copy(rb, accum.at[idst.at[j]], ssem, add=True)
        pltpu.make_async_copy(rows[0], accum.at[idst.at[J - 2]], ssem).wait()
        pltpu.make_async_copy(rows[1], accum.at[idst.at[J - 1]], ssem).wait()
        return carry
    lax.fori_loop(0, nchunk, chunk_body, 0)

    plsc.subcore_barrier()
    pltpu.sync_copy(accum.at[pl.ds(o0, RPT)],
                    out_hbm.at[pl.ds(c * N + o0, RPT)])

    @pl.when(s == NS - 1)
    def _():
        pltpu.sync_copy(accum.at[pl.ds(NS * RPT, TAIL)],
                        out_hbm.at[pl.ds(c * N + NS * RPT, TAIL)])


def _tc_dinv_body(dp_ref, dinv_ref):
    deg = jnp.sum(dp_ref[...], axis=0) + 1.0
    dinv_ref[...] = jnp.where(deg > 0.0, lax.rsqrt(deg), 0.0)


def _tc_pre_body(dinv_ref, x_ref, w_ref, xw_ref, u_ref):
    dinv = dinv_ref[0, 0, :]
    xw = jnp.dot(x_ref[...], w_ref[...], preferred_element_type=jnp.float32)
    xw_ref[...] = xw
    u_ref[...] = xw * dinv[:, None]


def _tc_mid_body(p_ref, xw_ref, dinv_ref, b_ref, w_ref, xw_out_ref, u_out_ref):
    dinv = dinv_ref[0, 0, :]
    agg = p_ref[0] + p_ref[1]
    h = jnp.maximum(
        agg * dinv[:, None] + xw_ref[...] * (dinv * dinv)[:, None]
        + b_ref[0][None, :], 0.0)
    xw2 = jnp.dot(h, w_ref[...], preferred_element_type=jnp.float32)
    xw_out_ref[...] = xw2
    u_out_ref[...] = xw2 * dinv[:, None]


def _tc_pool_body(p_ref, xw_ref, dinv_ref, b_ref, batch_ref, fcw_ref, fcb_ref,
                  out_ref, sums_ref, cnts_ref):
    i = pl.program_id(0)

    @pl.when(i == 0)
    def _():
        sums_ref[...] = jnp.zeros_like(sums_ref)
        cnts_ref[...] = jnp.zeros_like(cnts_ref)

    dinv = dinv_ref[0, 0, :]
    agg = p_ref[0] + p_ref[1]
    h = jnp.maximum(
        agg * dinv[:, None] + xw_ref[...] * (dinv * dinv)[:, None]
        + b_ref[0][None, :], 0.0)
    seg = batch_ref[0, 0, :]
    bn = seg.shape[0]
    onehot = (lax.broadcasted_iota(jnp.int32, (G, bn), 0)
              == seg[None, :]).astype(jnp.float32)
    sums_ref[...] += jnp.dot(onehot, h, preferred_element_type=jnp.float32)
    cnts_ref[...] += jnp.sum(onehot, axis=1, keepdims=True)

    @pl.when(i == pl.num_programs(0) - 1)
    def _():
        pooled = sums_ref[...] / jnp.maximum(cnts_ref[...], 1.0)
        res = jnp.dot(pooled, fcw_ref[...], preferred_element_type=jnp.float32)
        out_ref[...] = res[:, 0] + fcb_ref[0, 0]


def _mesh():
    return plsc.VectorSubcoreMesh(core_axis_name="c", subcore_axis_name="s",
                                  num_cores=NC, num_subcores=NS)


_SC_PARAMS = pltpu.CompilerParams(needs_layout_passes=False)


@jax.jit
def kernel(x, edge_index, edge_attr, batch, W1, b1, W2, b2, W3, b3, fcW, fcb):
    E = edge_attr.shape[0]
    unit = NW * J * 128
    epad = ((E + unit - 1) // unit) * unit
    pad = epad - E
    erows = epad // 128

    src = edge_index[0].astype(jnp.int32)
    dst = edge_index[1].astype(jnp.int32)
    ew = edge_attr.astype(jnp.float32)
    if pad:
        zi = jnp.zeros((pad,), jnp.int32)
        src = jnp.concatenate([src, zi])
        dst = jnp.concatenate([dst, zi])
        ew = jnp.concatenate([ew, jnp.zeros((pad,), jnp.float32)])
    src2 = src.reshape(erows, 128)
    dst2 = dst.reshape(erows, 128)
    ew2 = ew.reshape(erows, 128)

    sc_deg = pl.kernel(
        _sc_deg_body,
        out_type=jax.ShapeDtypeStruct((NW, 1, N), jnp.float32),
        mesh=_mesh(),
        scratch_types=[
            pltpu.VMEM((erows // NW, 128), jnp.float32),
            pltpu.VMEM((erows // NW, 128), jnp.int32),
            pltpu.VMEM((1, N), jnp.float32),
        ],
        compiler_params=_SC_PARAMS,
    )
    deg_partials = sc_deg(ew2, dst2)

    dinv2d = pl.pallas_call(
        _tc_dinv_body,
        grid=(1,),
        in_specs=[pl.BlockSpec((NW, 1, N), lambda i: (0, 0, 0))],
        out_specs=pl.BlockSpec((1, N), lambda i: (0, 0)),
        out_shape=jax.ShapeDtypeStruct((1, N), jnp.float32),
    )(deg_partials)

    nb = 10
    bn = N // nb
    dinv3 = dinv2d.reshape(nb, 1, bn)
    batch3 = batch.astype(jnp.int32).reshape(nb, 1, bn)

    row_spec = pl.BlockSpec((bn, H), lambda i: (i, 0))
    dinv_spec = pl.BlockSpec((1, 1, bn), lambda i: (i, 0, 0))
    w_spec = pl.BlockSpec((H, H), lambda i: (0, 0))
    b_spec = pl.BlockSpec((1, H), lambda i: (0, 0))
    p_spec = pl.BlockSpec((NC, bn, H), lambda i: (0, i, 0))
    mat_shape = jax.ShapeDtypeStruct((N, H), jnp.float32)

    tc_pre = pl.pallas_call(
        _tc_pre_body,
        grid=(nb,),
        in_specs=[dinv_spec, row_spec, w_spec],
        out_specs=[row_spec, row_spec],
        out_shape=[mat_shape, mat_shape],
    )

    tc_mid = pl.pallas_call(
        _tc_mid_body,
        grid=(nb,),
        in_specs=[p_spec, row_spec, dinv_spec, b_spec, w_spec],
        out_specs=[row_spec, row_spec],
        out_shape=[mat_shape, mat_shape],
    )

    sc_edges = pl.kernel(
        _sc_edges_body,
        out_type=jax.ShapeDtypeStruct((NC * N, H), jnp.float32),
        mesh=_mesh(),
        scratch_types=[
            pltpu.VMEM((J, 128), jnp.int32),
            pltpu.VMEM((J, 128), jnp.int32),
            pltpu.VMEM((J, 128), jnp.float32),
            pltpu.VMEM((128, H), jnp.float32),
            pltpu.VMEM((128, H), jnp.float32),
            pltpu.VMEM_SHARED((N, H), jnp.float32),
            pltpu.SemaphoreType.DMA,
            pltpu.SemaphoreType.DMA,
        ],
        compiler_params=_SC_PARAMS,
    )

    xw1, u1 = tc_pre(dinv3, x, W1)
    p1 = sc_edges(u1, src2, dst2, ew2).reshape(NC, N, H)
    xw2, u2 = tc_mid(p1, xw1, dinv3, b1.reshape(1, H), W2)
    p2 = sc_edges(u2, src2, dst2, ew2).reshape(NC, N, H)
    xw3, u3 = tc_mid(p2, xw2, dinv3, b2.reshape(1, H), W3)
    p3 = sc_edges(u3, src2, dst2, ew2).reshape(NC, N, H)

    out, _sums, _cnts = pl.pallas_call(
        _tc_pool_body,
        grid=(nb,),
        in_specs=[p_spec, row_spec, dinv_spec, b_spec,
                  pl.BlockSpec((1, 1, bn), lambda i: (i, 0, 0)),
                  pl.BlockSpec((H, 1), lambda i: (0, 0)),
                  pl.BlockSpec((1, 1), lambda i: (0, 0))],
        out_specs=[pl.BlockSpec((G,), lambda i: (0,)),
                   pl.BlockSpec((G, H), lambda i: (0, 0)),
                   pl.BlockSpec((G, 1), lambda i: (0, 0))],
        out_shape=[jax.ShapeDtypeStruct((G,), jnp.float32),
                   jax.ShapeDtypeStruct((G, H), jnp.float32),
                   jax.ShapeDtypeStruct((G, 1), jnp.float32)],
    )(p3, xw3, dinv3, b3.reshape(1, H), batch3, fcW, fcb.reshape(1, 1))

    return out
